# baseline (speedup 1.0000x reference)
import functools

import jax
import jax.numpy as jnp
from jax import lax
from jax.experimental import pallas as pl
from jax.experimental.pallas import tpu as pltpu
from jax.experimental.pallas import tpu_sc as plsc

B = 16384
FW = 20
ED = 32
VOCAB = 1000000
NW = 32
ROWS_W = B // NW
R = 32
NCHUNK = ROWS_W // R
GPC = R * FW // 128

LC = 512
NTR = (VOCAB // 128) * 128
TCH = NTR // LC
NK = 62
TAIL = VOCAB - NTR

def _lane():
    return lax.broadcasted_iota(jnp.int32, (16,), 0)


def _bit(k):
    return (_lane() >> k) & 1


def _xsh(v, k):
    return v.at[_lane() ^ (1 << k)].get(mode="promise_in_bounds")


def _transpose16(vs):
    for k in range(4):
        s = 1 << k
        nv = list(vs)
        for j in range(16):
            if j & s == 0:
                a, b = vs[j], vs[j + s]
                nv[j] = jnp.where(_bit(k) == 0, a, _xsh(b, k))
                nv[j + s] = jnp.where(_bit(k) == 0, _xsh(a, k), b)
        vs = nv
    return vs


def _tr_body(tabt_hbm, tail_hbm, rm_hbm, ib0, ib1, ob0, ob1, tb,
             si0, si1, so0, so1):
    cid = lax.axis_index("c")
    sid = lax.axis_index("s")
    wid = sid * 2 + cid
    def fire_in(k, ibuf, sem):
        ci = wid + 32 * k
        pltpu.async_copy(tabt_hbm.at[:, pl.ds(ci * LC, LC)], ibuf, sem)

    def drain_in(ibuf, sem):
        pltpu.make_async_copy(tabt_hbm.at[:, pl.ds(0, LC)], ibuf, sem).wait()

    def fire_out(k, obuf, sem):
        ci = wid + 32 * k
        pltpu.async_copy(obuf, rm_hbm.at[pl.ds(ci * (LC // 4), LC // 4), :],
                         sem)

    def drain_out(obuf, sem):
        pltpu.make_async_copy(rm_hbm.at[pl.ds(0, LC // 4), :], obuf,
                              sem).wait()

    def compute(ibuf, obuf):
        def i_body(g, carry):
            i0 = g * 16
            lo = _transpose16([ibuf[e, pl.ds(i0, 16)] for e in range(16)])
            hi = _transpose16([ibuf[e + 16, pl.ds(i0, 16)]
                               for e in range(16)])
            for j in range(16):
                r = g * 4 + j // 4
                off = (j % 4) * 32
                obuf[r, pl.ds(off, 16)] = lo[j]
                obuf[r, pl.ds(off + 16, 16)] = hi[j]
            return carry

        lax.fori_loop(0, LC // 16, i_body, 0)

    def valid(k):
        return (wid + 32 * k) < TCH

    fire_in(0, ib0, si0)

    def loop_body(t, carry):
        k0 = 2 * t
        k1 = 2 * t + 1

        @pl.when(valid(k1))
        def _():
            fire_in(k1, ib1, si1)

        drain_in(ib0, si0)

        @pl.when(t > 0)
        def _():
            drain_out(ob0, so0)

        compute(ib0, ob0)
        fire_out(k0, ob0, so0)

        @pl.when(valid(k0 + 2))
        def _():
            fire_in(k0 + 2, ib0, si0)

        @pl.when(valid(k1))
        def _():
            drain_in(ib1, si1)

            @pl.when(t > 0)
            def _():
                drain_out(ob1, so1)

            compute(ib1, ob1)
            fire_out(k1, ob1, so1)

        return carry

    lax.fori_loop(0, NK // 2, loop_body, 0)
    drain_out(ob0, so0)

    @pl.when(valid(NK - 1))
    def _():
        drain_out(ob1, so1)

    @pl.when(wid == 31)
    def _():
        pltpu.sync_copy(tail_hbm, tb)
        for j in range(TAIL // 4):
            for k in range(4):
                ob0[j, pl.ds(k * 32, 16)] = tb[4 * j + k, pl.ds(0, 16)]
                ob0[j, pl.ds(k * 32 + 16, 16)] = tb[4 * j + k, pl.ds(16, 16)]
        pltpu.sync_copy(ob0.at[pl.ds(0, TAIL // 4), :],
                        rm_hbm.at[pl.ds(NTR // 4, TAIL // 4), :])


_tr_call = functools.partial(
    pl.kernel,
    out_type=jax.ShapeDtypeStruct((VOCAB // 4, 128), jnp.float32),
    mesh=plsc.VectorSubcoreMesh(core_axis_name="c", subcore_axis_name="s"),
    compiler_params=pltpu.CompilerParams(use_tc_tiling_on_sc=True),
    scratch_types=[
        pltpu.VMEM((ED, LC), jnp.float32),
        pltpu.VMEM((ED, LC), jnp.float32),
        pltpu.VMEM((LC // 4, 128), jnp.float32),
        pltpu.VMEM((LC // 4, 128), jnp.float32),
        pltpu.VMEM((TAIL, ED), jnp.float32),
        pltpu.SemaphoreType.DMA,
        pltpu.SemaphoreType.DMA,
        pltpu.SemaphoreType.DMA,
        pltpu.SemaphoreType.DMA,
    ],
)(_tr_body)


def _drain(tab_hbm, dbuf, sem):
    pltpu.make_async_copy(tab_hbm.at[pl.ds(0, R * FW)], dbuf, sem).wait()


def _sc_body(x_hbm, tab_hbm, w_hbm, b_hbm, out_hbm,
             idx0, idx1, idx2, idx3, d0, d1, d2, d3,
             w_v, b_v, out_v, sem0, sem1, sem2, sem3):
    cid = lax.axis_index("c")
    sid = lax.axis_index("s")
    wid = sid * 2 + cid
    base = wid * ROWS_W

    pltpu.sync_copy(w_hbm, w_v)
    pltpu.sync_copy(b_hbm, b_v.at[pl.ds(0, 1)])
    bs = b_v[pl.ds(0, 16)][0]

    def stage_fire(cix, idxbuf, dbuf, sem):
        pltpu.sync_copy(
            x_hbm.at[pl.ds((base * FW) + cix * (R * FW), R * FW)], idxbuf)
        pltpu.async_copy(tab_hbm.at[idxbuf], dbuf, sem)

    def compute(cix, dbuf):
        def blk_body(blk, carry):
            accs = [jnp.zeros((16,), jnp.float32) for _ in range(16)]
            rbase = blk * (16 * FW)
            for k in range(2 * FW):
                wk = w_v[pl.ds(k * 16, 16)]
                rh = k // 2
                off = (k % 2) * 16
                for j in range(16):
                    d = dbuf[rbase + j * FW + rh, pl.ds(off, 16)]
                    accs[j] = accs[j] + d * wk
            vs = accs
            for k in range(4):
                nxt = []
                for p in range(len(vs) // 2):
                    a, b = vs[2 * p], vs[2 * p + 1]
                    nxt.append(jnp.where(_bit(k) == 0,
                                         a + _xsh(a, k), b + _xsh(b, k)))
                vs = nxt
            out_v[pl.ds(blk * 16, 16)] = jnp.exp(vs[0] + bs)
            return carry

        lax.fori_loop(0, R // 16, blk_body, 0)
        pltpu.sync_copy(out_v, out_hbm.at[pl.ds(base + cix * R, R)])

    bufs = [(idx0, d0, sem0), (idx1, d1, sem1), (idx2, d2, sem2),
            (idx3, d3, sem3)]
    for p in range(3):
        stage_fire(p, *bufs[p])

    def loop_body(t, carry):
        for p in range(4):
            k = 4 * t + p
            ib, db, sm = bufs[p]

            @pl.when(k + 3 < NCHUNK)
            def _():
                ib3, db3, sm3 = bufs[(p + 3) % 4]
                stage_fire(k + 3, ib3, db3, sm3)

            _drain(tab_hbm, db, sm)
            compute(k, db)
        return carry

    lax.fori_loop(0, NCHUNK // 4, loop_body, 0)


_sc_call = functools.partial(
    pl.kernel,
    out_type=jax.ShapeDtypeStruct((B,), jnp.float32),
    mesh=plsc.VectorSubcoreMesh(core_axis_name="c", subcore_axis_name="s"),
    compiler_params=pltpu.CompilerParams(use_tc_tiling_on_sc=False),
    scratch_types=[
        pltpu.VMEM((R * FW,), jnp.int32),
        pltpu.VMEM((R * FW,), jnp.int32),
        pltpu.VMEM((R * FW,), jnp.int32),
        pltpu.VMEM((R * FW,), jnp.int32),
        pltpu.VMEM((R * FW, ED), jnp.float32),
        pltpu.VMEM((R * FW, ED), jnp.float32),
        pltpu.VMEM((R * FW, ED), jnp.float32),
        pltpu.VMEM((R * FW, ED), jnp.float32),
        pltpu.VMEM((ED * FW,), jnp.float32),
        pltpu.VMEM((16,), jnp.float32),
        pltpu.VMEM((R,), jnp.float32),
        pltpu.SemaphoreType.DMA,
        pltpu.SemaphoreType.DMA,
        pltpu.SemaphoreType.DMA,
        pltpu.SemaphoreType.DMA,
    ],
)(_sc_body)


@jax.jit
def kernel(x, table, W, b):
    x2 = x.astype(jnp.int32).reshape(B * FW)
    rm128 = _tr_call(table.T, table[NTR:, :])
    rm = rm128.reshape(VOCAB, ED)
    return _sc_call(x2, rm, W.reshape(ED * FW), b)


# speedup vs baseline: 1.1073x; 1.1073x over previous
import functools

import jax
import jax.numpy as jnp
from jax import lax
from jax.experimental import pallas as pl
from jax.experimental.pallas import tpu as pltpu
from jax.experimental.pallas import tpu_sc as plsc

B = 16384
FW = 20
ED = 32
VOCAB = 1000000
NW = 32
ROWS_W = B // NW
R = 64
NCHUNK = ROWS_W // R
GPC = R * FW // 128

LC = 512
NTR = (VOCAB // 128) * 128
TCH = NTR // LC
NK = 62
TAIL = VOCAB - NTR

def _lane():
    return lax.broadcasted_iota(jnp.int32, (16,), 0)


def _bit(k):
    return (_lane() >> k) & 1


def _xsh(v, k):
    return v.at[_lane() ^ (1 << k)].get(mode="promise_in_bounds")


def _transpose16(vs):
    for k in range(4):
        s = 1 << k
        nv = list(vs)
        for j in range(16):
            if j & s == 0:
                a, b = vs[j], vs[j + s]
                nv[j] = jnp.where(_bit(k) == 0, a, _xsh(b, k))
                nv[j + s] = jnp.where(_bit(k) == 0, _xsh(a, k), b)
        vs = nv
    return vs


def _tr_body(tabt_hbm, tail_hbm, rm_hbm, ib0, ib1, ob0, ob1, tb,
             si0, si1, so0, so1):
    cid = lax.axis_index("c")
    sid = lax.axis_index("s")
    wid = sid * 2 + cid
    def fire_in(k, ibuf, sem):
        ci = wid + 32 * k
        pltpu.async_copy(tabt_hbm.at[:, pl.ds(ci * LC, LC)], ibuf, sem)

    def drain_in(ibuf, sem):
        pltpu.make_async_copy(tabt_hbm.at[:, pl.ds(0, LC)], ibuf, sem).wait()

    def fire_out(k, obuf, sem):
        ci = wid + 32 * k
        pltpu.async_copy(obuf, rm_hbm.at[pl.ds(ci * (LC // 4), LC // 4), :],
                         sem)

    def drain_out(obuf, sem):
        pltpu.make_async_copy(rm_hbm.at[pl.ds(0, LC // 4), :], obuf,
                              sem).wait()

    def compute(ibuf, obuf):
        def i_body(g, carry):
            i0 = g * 16
            lo = _transpose16([ibuf[e, pl.ds(i0, 16)] for e in range(16)])
            hi = _transpose16([ibuf[e + 16, pl.ds(i0, 16)]
                               for e in range(16)])
            for j in range(16):
                r = g * 4 + j // 4
                off = (j % 4) * 32
                obuf[r, pl.ds(off, 16)] = lo[j]
                obuf[r, pl.ds(off + 16, 16)] = hi[j]
            return carry

        lax.fori_loop(0, LC // 16, i_body, 0)

    def valid(k):
        return (wid + 32 * k) < TCH

    fire_in(0, ib0, si0)

    def loop_body(t, carry):
        k0 = 2 * t
        k1 = 2 * t + 1

        @pl.when(valid(k1))
        def _():
            fire_in(k1, ib1, si1)

        drain_in(ib0, si0)

        @pl.when(t > 0)
        def _():
            drain_out(ob0, so0)

        compute(ib0, ob0)
        fire_out(k0, ob0, so0)

        @pl.when(valid(k0 + 2))
        def _():
            fire_in(k0 + 2, ib0, si0)

        @pl.when(valid(k1))
        def _():
            drain_in(ib1, si1)

            @pl.when(t > 0)
            def _():
                drain_out(ob1, so1)

            compute(ib1, ob1)
            fire_out(k1, ob1, so1)

        return carry

    lax.fori_loop(0, NK // 2, loop_body, 0)
    drain_out(ob0, so0)

    @pl.when(valid(NK - 1))
    def _():
        drain_out(ob1, so1)

    @pl.when(wid == 31)
    def _():
        pltpu.sync_copy(tail_hbm, tb)
        for j in range(TAIL // 4):
            for k in range(4):
                ob0[j, pl.ds(k * 32, 16)] = tb[4 * j + k, pl.ds(0, 16)]
                ob0[j, pl.ds(k * 32 + 16, 16)] = tb[4 * j + k, pl.ds(16, 16)]
        pltpu.sync_copy(ob0.at[pl.ds(0, TAIL // 4), :],
                        rm_hbm.at[pl.ds(NTR // 4, TAIL // 4), :])


_tr_call = functools.partial(
    pl.kernel,
    out_type=jax.ShapeDtypeStruct((VOCAB // 4, 128), jnp.float32),
    mesh=plsc.VectorSubcoreMesh(core_axis_name="c", subcore_axis_name="s"),
    compiler_params=pltpu.CompilerParams(use_tc_tiling_on_sc=True),
    scratch_types=[
        pltpu.VMEM((ED, LC), jnp.float32),
        pltpu.VMEM((ED, LC), jnp.float32),
        pltpu.VMEM((LC // 4, 128), jnp.float32),
        pltpu.VMEM((LC // 4, 128), jnp.float32),
        pltpu.VMEM((TAIL, ED), jnp.float32),
        pltpu.SemaphoreType.DMA,
        pltpu.SemaphoreType.DMA,
        pltpu.SemaphoreType.DMA,
        pltpu.SemaphoreType.DMA,
    ],
)(_tr_body)


def _drain(tab_hbm, dbuf, sem):
    pltpu.make_async_copy(tab_hbm.at[pl.ds(0, R * FW)], dbuf, sem).wait()


def _sc_body(x_hbm, tab_hbm, w_hbm, b_hbm, out_hbm,
             idx0, idx1, d0, d1, w_v, b_v, out_v, sem0, sem1):
    cid = lax.axis_index("c")
    sid = lax.axis_index("s")
    wid = sid * 2 + cid
    base = wid * ROWS_W

    pltpu.sync_copy(w_hbm, w_v)
    pltpu.sync_copy(b_hbm, b_v.at[pl.ds(0, 1)])
    bs = b_v[pl.ds(0, 16)][0]

    def stage_fire(cix, idxbuf, dbuf, sem):
        pltpu.sync_copy(
            x_hbm.at[pl.ds((base * FW) + cix * (R * FW), R * FW)], idxbuf)
        pltpu.async_copy(tab_hbm.at[idxbuf], dbuf, sem)

    def compute(cix, dbuf):
        def blk_body(blk, carry):
            accs = [jnp.zeros((16,), jnp.float32) for _ in range(16)]
            rbase = blk * (16 * FW)
            for k in range(2 * FW):
                wk = w_v[pl.ds(k * 16, 16)]
                rh = k // 2
                off = (k % 2) * 16
                for j in range(16):
                    d = dbuf[rbase + j * FW + rh, pl.ds(off, 16)]
                    accs[j] = accs[j] + d * wk
            vs = accs
            for k in range(4):
                nxt = []
                for p in range(len(vs) // 2):
                    a, b = vs[2 * p], vs[2 * p + 1]
                    nxt.append(jnp.where(_bit(k) == 0,
                                         a + _xsh(a, k), b + _xsh(b, k)))
                vs = nxt
            out_v[pl.ds(blk * 16, 16)] = jnp.exp(vs[0] + bs)
            return carry

        lax.fori_loop(0, R // 16, blk_body, 0)
        pltpu.sync_copy(out_v, out_hbm.at[pl.ds(base + cix * R, R)])

    stage_fire(0, idx0, d0, sem0)

    def loop_body(t, carry):
        c0 = 2 * t
        stage_fire(c0 + 1, idx1, d1, sem1)
        _drain(tab_hbm, d0, sem0)
        compute(c0, d0)

        @pl.when(t < NCHUNK // 2 - 1)
        def _():
            stage_fire(c0 + 2, idx0, d0, sem0)

        _drain(tab_hbm, d1, sem1)
        compute(c0 + 1, d1)
        return carry

    lax.fori_loop(0, NCHUNK // 2, loop_body, 0)


_sc_call = functools.partial(
    pl.kernel,
    out_type=jax.ShapeDtypeStruct((B,), jnp.float32),
    mesh=plsc.VectorSubcoreMesh(core_axis_name="c", subcore_axis_name="s"),
    compiler_params=pltpu.CompilerParams(use_tc_tiling_on_sc=False),
    scratch_types=[
        pltpu.VMEM((R * FW,), jnp.int32),
        pltpu.VMEM((R * FW,), jnp.int32),
        pltpu.VMEM((R * FW, ED), jnp.float32),
        pltpu.VMEM((R * FW, ED), jnp.float32),
        pltpu.VMEM((ED * FW,), jnp.float32),
        pltpu.VMEM((16,), jnp.float32),
        pltpu.VMEM((R,), jnp.float32),
        pltpu.SemaphoreType.DMA,
        pltpu.SemaphoreType.DMA,
    ],
)(_sc_body)


@jax.jit
def kernel(x, table, W, b):
    x2 = x.astype(jnp.int32).reshape(B * FW)
    rm128 = _tr_call(table.T, table[NTR:, :])
    rm = rm128.reshape(VOCAB, ED)
    return _sc_call(x2, rm, W.reshape(ED * FW), b)
